# baseline (speedup 1.0000x reference)
import functools

import jax
import jax.numpy as jnp
from jax import lax
from jax.experimental import pallas as pl
from jax.experimental.pallas import tpu as pltpu
from jax.experimental.pallas import tpu_sc as plsc

PAD_ROW = 0
V = 1000
VP = 1024
VW = VP // 2
H = 64
B = 1024
L = 50
LP = 56

NC = 2
NS = 16
NW = NC * NS
BPW = B // NW
TPW = BPW * LP


def _proj_body(emb_ref, w_ref, b_ref, out_ref):
    emb = emb_ref[:]
    rows = lax.broadcasted_iota(jnp.int32, emb.shape, 0)
    emb = jnp.where(rows == PAD_ROW, 0.0, emb)
    acc = lax.dot_general(
        emb, w_ref[:], (((1,), (1,)), ((), ())),
        preferred_element_type=jnp.float32,
    )
    out_ref[:] = acc + b_ref[:]


def _logit_table(emb, w, b):
    wp = jnp.concatenate([w, jnp.zeros((VP - V, H), w.dtype)], axis=0)
    bp = jnp.concatenate([b, jnp.zeros((VP - V,), b.dtype)])
    return pl.pallas_call(
        _proj_body,
        out_shape=jax.ShapeDtypeStruct((V, VP), jnp.float32),
    )(emb, wp, bp.reshape(1, VP))


def _pack_table(lt):
    bits = lax.bitcast_convert_type(lt.astype(jnp.bfloat16), jnp.uint16)
    quads = bits.reshape(V, 4, 2, 128).astype(jnp.uint32)
    words = quads[:, :, 0, :] | (quads[:, :, 1, :] << 16)
    return lax.bitcast_convert_type(words, jnp.int32).reshape(V, VW)


def _unpack_rows(g_ref, f_ref, src0, nrows):
    def row(r, carry):
        for sp in range(4):
            for h in range(8):
                w = g_ref[src0 + r, pl.ds(128 * sp + 16 * h, 16)]
                lo = lax.bitcast_convert_type(lax.shift_left(w, 16), jnp.float32)
                f_ref[r, pl.ds(256 * sp + 16 * h, 16)] = lo
                hicol = 256 * sp + 128 + 16 * h
                if hicol + 16 <= V:
                    hi = lax.bitcast_convert_type(w & jnp.int32(-65536), jnp.float32)
                    f_ref[r, pl.ds(hicol, 16)] = hi
        wt = g_ref[src0 + r, pl.ds(128 * 3 + 88, 16)]
        f_ref[r, pl.ds(V - 16, 16)] = lax.bitcast_convert_type(
            wt & jnp.int32(-65536), jnp.float32)
        return carry
    lax.fori_loop(0, nrows, row, 0)


def _gather_body(lt_hbm, ids_hbm, out_hbm, idx_v, g1, g2, f1, f2, ft,
                 gsem, osem):
    c = lax.axis_index("c")
    s = lax.axis_index("s")
    wid = s * NC + c
    tok0 = wid * TPW
    b0 = wid * BPW

    pltpu.sync_copy(ids_hbm.at[pl.ds(tok0, TPW)], idx_v)

    def c1_dst(bb):
        return out_hbm.at[bb].at[pl.ds(0, 32)]

    def c2_dst(bb):
        return out_hbm.at[bb].at[pl.ds(32, 16)]

    def ct_dst(bb):
        return out_hbm.at[bb].at[pl.ds(48, 2)]

    def step(i, carry):
        bb = b0 + i
        t0 = i * LP

        pltpu.async_copy(lt_hbm.at[idx_v.at[pl.ds(t0, 32)]], g1, gsem)
        pltpu.async_copy(lt_hbm.at[idx_v.at[pl.ds(t0 + 32, 24)]], g2, gsem)

        @pl.when(i >= 1)
        def _():
            pltpu.make_async_copy(f1, c1_dst(bb - 1), osem).wait()
        pltpu.make_async_copy(lt_hbm.at[idx_v.at[pl.ds(t0, 32)]],
                              g1, gsem).wait()
        _unpack_rows(g1, f1, 0, 32)
        pltpu.async_copy(f1, c1_dst(bb), osem)

        @pl.when(i >= 1)
        def _():
            pltpu.make_async_copy(
                f2.at[pl.ds(0, 16)], c2_dst(bb - 1), osem).wait()
            pltpu.make_async_copy(ft, ct_dst(bb - 1), osem).wait()
        pltpu.make_async_copy(lt_hbm.at[idx_v.at[pl.ds(t0 + 32, 24)]],
                              g2, gsem).wait()
        _unpack_rows(g2, f2, 0, 16)
        _unpack_rows(g2, ft, 16, 2)
        pltpu.async_copy(f2.at[pl.ds(0, 16)], c2_dst(bb), osem)
        pltpu.async_copy(ft, ct_dst(bb), osem)
        return carry

    lax.fori_loop(0, BPW, step, 0)
    blast = b0 + BPW - 1
    pltpu.make_async_copy(f1, c1_dst(blast), osem).wait()
    pltpu.make_async_copy(f2.at[pl.ds(0, 16)], c2_dst(blast), osem).wait()
    pltpu.make_async_copy(ft, ct_dst(blast), osem).wait()


_gather = functools.partial(
    pl.kernel,
    out_type=jax.ShapeDtypeStruct((B, L, V), jnp.float32),
    mesh=plsc.VectorSubcoreMesh(core_axis_name="c", subcore_axis_name="s"),
    scratch_types=[
        pltpu.VMEM((TPW,), jnp.int32),
        pltpu.VMEM((32, VW), jnp.int32),
        pltpu.VMEM((24, VW), jnp.int32),
        pltpu.VMEM((32, V), jnp.float32),
        pltpu.VMEM((16, V), jnp.float32),
        pltpu.VMEM((2, V), jnp.float32),
        pltpu.SemaphoreType.DMA,
        pltpu.SemaphoreType.DMA,
    ],
)(_gather_body)


def kernel(token_ids, emb_table, W, b):
    ltp = _pack_table(_logit_table(emb_table, W, b))
    ids_pad = jnp.pad(token_ids, ((0, 0), (0, LP - L))).reshape(-1)
    return _gather(ltp, ids_pad)


# speedup vs baseline: 2.0656x; 2.0656x over previous
import functools

import jax
import jax.numpy as jnp
from jax import lax
from jax.experimental import pallas as pl
from jax.experimental.pallas import tpu as pltpu

PAD_ROW = 0
V = 1000
VP = 1024
H = 64
B = 1024
L = 50
LP = 56
BT = 8
GRID = B // BT


def _proj_body(emb_ref, w_ref, b_ref, out_ref):
    emb = emb_ref[:]
    rows = lax.broadcasted_iota(jnp.int32, emb.shape, 0)
    emb = jnp.where(rows == PAD_ROW, 0.0, emb)
    acc = lax.dot_general(
        emb, w_ref[:], (((1,), (1,)), ((), ())),
        preferred_element_type=jnp.float32,
    )
    out_ref[:] = acc + b_ref[:]


def _logit_table(emb, w, b):
    return pl.pallas_call(
        _proj_body,
        out_shape=jax.ShapeDtypeStruct((V, V), jnp.float32),
    )(emb, w, b.reshape(1, V))


def _onehot_body(ids_ref, lt_ref, out_ref):
    ids = ids_ref[:]
    vocab = lax.broadcasted_iota(jnp.int32, (BT, LP, VP), 2)
    onehot = (ids[:, :, None] == vocab).astype(jnp.bfloat16)
    acc = lax.dot_general(
        onehot, lt_ref[:], (((2,), (0,)), ((), ())),
        preferred_element_type=jnp.float32,
    )
    out_ref[:] = acc[:, :L, :]


def _onehot_gather(ids_pad, ltb):
    return pl.pallas_call(
        _onehot_body,
        grid=(GRID,),
        in_specs=[
            pl.BlockSpec((BT, LP), lambda i: (i, 0)),
            pl.BlockSpec((VP, V), lambda i: (0, 0)),
        ],
        out_specs=pl.BlockSpec((BT, L, V), lambda i: (i, 0, 0)),
        out_shape=jax.ShapeDtypeStruct((B, L, V), jnp.float32),
    )(ids_pad, ltb)


def kernel(token_ids, emb_table, W, b):
    lt = _logit_table(emb_table, W, b)
    ltb = jnp.concatenate(
        [lt, jnp.zeros((VP - V, V), lt.dtype)], axis=0).astype(jnp.bfloat16)
    ids_pad = jnp.pad(token_ids, ((0, 0), (0, LP - L)))
    return _onehot_gather(ids_pad, ltb)


# speedup vs baseline: 2.1711x; 1.0511x over previous
import functools

import jax
import jax.numpy as jnp
from jax import lax
from jax.experimental import pallas as pl
from jax.experimental.pallas import tpu as pltpu

PAD_ROW = 0
V = 1000
VP = 1024
H = 64
B = 1024
L = 50
LP = 56
BT = 16
GRID = B // BT


def _proj_body(emb_ref, w_ref, b_ref, out_ref):
    emb = emb_ref[:]
    rows = lax.broadcasted_iota(jnp.int32, emb.shape, 0)
    emb = jnp.where(rows == PAD_ROW, 0.0, emb)
    acc = lax.dot_general(
        emb, w_ref[:], (((1,), (1,)), ((), ())),
        preferred_element_type=jnp.float32,
    )
    out_ref[:] = acc + b_ref[:]


def _logit_table(emb, w, b):
    return pl.pallas_call(
        _proj_body,
        out_shape=jax.ShapeDtypeStruct((V, V), jnp.float32),
    )(emb, w, b.reshape(1, V))


def _onehot_body(ids_ref, lt_ref, out_ref):
    ids = ids_ref[:]
    vocab = lax.broadcasted_iota(jnp.int32, (BT, LP, VP), 2)
    onehot = (ids[:, :, None] == vocab).astype(jnp.bfloat16)
    acc = lax.dot_general(
        onehot, lt_ref[:], (((2,), (0,)), ((), ())),
        preferred_element_type=jnp.float32,
    )
    out_ref[:] = acc[:, :L, :]


def _onehot_gather(ids_pad, ltb):
    return pl.pallas_call(
        _onehot_body,
        grid=(GRID,),
        in_specs=[
            pl.BlockSpec((BT, LP), lambda i: (i, 0)),
            pl.BlockSpec((VP, V), lambda i: (0, 0)),
        ],
        out_specs=pl.BlockSpec((BT, L, V), lambda i: (i, 0, 0)),
        out_shape=jax.ShapeDtypeStruct((B, L, V), jnp.float32),
    )(ids_pad, ltb)


def kernel(token_ids, emb_table, W, b):
    lt = _logit_table(emb_table, W, b)
    ltb = jnp.concatenate(
        [lt, jnp.zeros((VP - V, V), lt.dtype)], axis=0).astype(jnp.bfloat16)
    ids_pad = jnp.pad(token_ids, ((0, 0), (0, LP - L)))
    return _onehot_gather(ids_pad, ltb)


# speedup vs baseline: 2.1784x; 1.0034x over previous
import functools

import jax
import jax.numpy as jnp
from jax import lax
from jax.experimental import pallas as pl
from jax.experimental.pallas import tpu as pltpu

PAD_ROW = 0
V = 1000
VP = 1024
H = 64
B = 1024
L = 50
LP = 56
BT = 32
GRID = B // BT


def _proj_body(emb_ref, w_ref, b_ref, out_ref):
    emb = emb_ref[:]
    rows = lax.broadcasted_iota(jnp.int32, emb.shape, 0)
    emb = jnp.where(rows == PAD_ROW, 0.0, emb)
    acc = lax.dot_general(
        emb, w_ref[:], (((1,), (1,)), ((), ())),
        preferred_element_type=jnp.float32,
    )
    out_ref[:] = acc + b_ref[:]


def _logit_table(emb, w, b):
    return pl.pallas_call(
        _proj_body,
        out_shape=jax.ShapeDtypeStruct((V, V), jnp.float32),
    )(emb, w, b.reshape(1, V))


def _onehot_body(ids_ref, lt_ref, out_ref):
    ids = ids_ref[:]
    vocab = lax.broadcasted_iota(jnp.int32, (BT, LP, VP), 2)
    onehot = (ids[:, :, None] == vocab).astype(jnp.bfloat16)
    acc = lax.dot_general(
        onehot, lt_ref[:], (((2,), (0,)), ((), ())),
        preferred_element_type=jnp.float32,
    )
    out_ref[:] = acc[:, :L, :]


def _onehot_gather(ids_pad, ltb):
    return pl.pallas_call(
        _onehot_body,
        grid=(GRID,),
        in_specs=[
            pl.BlockSpec((BT, LP), lambda i: (i, 0)),
            pl.BlockSpec((VP, V), lambda i: (0, 0)),
        ],
        out_specs=pl.BlockSpec((BT, L, V), lambda i: (i, 0, 0)),
        out_shape=jax.ShapeDtypeStruct((B, L, V), jnp.float32),
    )(ids_pad, ltb)


def kernel(token_ids, emb_table, W, b):
    lt = _logit_table(emb_table, W, b)
    ltb = jnp.concatenate(
        [lt, jnp.zeros((VP - V, V), lt.dtype)], axis=0).astype(jnp.bfloat16)
    ids_pad = jnp.pad(token_ids, ((0, 0), (0, LP - L)))
    return _onehot_gather(ids_pad, ltb)
